# baseline (speedup 1.0000x reference)
import functools

import jax
import jax.numpy as jnp
from jax import lax
from jax.experimental import pallas as pl
from jax.experimental.pallas import tpu as pltpu
from jax.experimental.pallas import tpu_sc as plsc

N = 10000
E = 320000
D = 128
H = 128
OUT = 64
G = 128

NC = 2
NS = 16
NW = NC * NS
K = 128
NCHUNK = 80
EPW = NCHUNK * K
E_PAD = NW * EPW
HCHUNK = NCHUNK // 2
ACC_N = 10240
DUMMY = ACC_N - 1
RPT = ACC_N // NS
ZK = 128
RB = 1000
NRB = N // RB


_mesh = plsc.VectorSubcoreMesh(core_axis_name="c", subcore_axis_name="s")


@functools.partial(
    pl.kernel,
    out_type=jax.ShapeDtypeStruct((NC, N, D), jnp.float32),
    mesh=_mesh,
    scratch_types=[
        pltpu.VMEM((HCHUNK, K), jnp.int32),
        pltpu.VMEM((HCHUNK, K), jnp.int32),
        pltpu.VMEM((K, D), jnp.float32),
        pltpu.VMEM((K, D), jnp.float32),
        pltpu.VMEM_SHARED((ACC_N, D), jnp.float32),
        pltpu.SemaphoreType.DMA,
        pltpu.SemaphoreType.DMA,
    ],
)
def _sc_agg(x_hbm, src_hbm, dst_hbm, zero_hbm, out_hbm,
            src_v, dst_v, rows_v, rows2_v, acc_sh, sem, sem2):
    c = lax.axis_index("c")
    s = lax.axis_index("s")
    wid = c * NS + s

    pltpu.sync_copy(zero_hbm, rows_v)
    for t in range(RPT // ZK):
        pltpu.sync_copy(rows_v, acc_sh.at[pl.ds(s * RPT + t * ZK, ZK)])
    plsc.subcore_barrier()

    rowsA = rows_v
    rowsB = rows2_v

    def chunk2(jj, carry):
        j0 = jj * 2
        j1 = j0 + 1
        pltpu.async_copy(x_hbm.at[src_v.at[j0]], rowsA, sem).wait()
        pltpu.sync_copy(rowsA, acc_sh.at[dst_v.at[j0]], add=True)
        pltpu.async_copy(x_hbm.at[src_v.at[j1]], rowsB, sem2).wait()
        pltpu.sync_copy(rowsB, acc_sh.at[dst_v.at[j1]], add=True)
        return carry

    for half in range(2):
        pltpu.sync_copy(src_hbm.at[wid].at[pl.ds(half * HCHUNK, HCHUNK)],
                        src_v)
        pltpu.sync_copy(dst_hbm.at[wid].at[pl.ds(half * HCHUNK, HCHUNK)],
                        dst_v)
        lax.fori_loop(0, HCHUNK // 2, chunk2, 0)
    plsc.subcore_barrier()

    @pl.when(s < NS - 1)
    def _():
        pltpu.sync_copy(acc_sh.at[pl.ds(s * RPT, RPT)],
                        out_hbm.at[c].at[pl.ds(s * RPT, RPT)])

    @pl.when(s == NS - 1)
    def _():
        tail = N - (NS - 1) * RPT
        pltpu.sync_copy(acc_sh.at[pl.ds((NS - 1) * RPT, tail)],
                        out_hbm.at[c].at[pl.ds((NS - 1) * RPT, tail)])


def _mlp_body(x_ref, agg_ref, wa_ref, ba_ref, wb_ref, bb_ref, o_ref):
    h = x_ref[...] + agg_ref[0] + agg_ref[1]
    h = jnp.maximum(
        jnp.dot(h, wa_ref[...], preferred_element_type=jnp.float32)
        + ba_ref[...], 0.0)
    h = jnp.maximum(
        jnp.dot(h, wb_ref[...], preferred_element_type=jnp.float32)
        + bb_ref[...], 0.0)
    o_ref[...] = h


_mlp_call = pl.pallas_call(
    _mlp_body,
    grid=(NRB,),
    in_specs=[
        pl.BlockSpec((RB, D), lambda i: (i, 0)),
        pl.BlockSpec((NC, RB, D), lambda i: (0, i, 0)),
        pl.BlockSpec((D, H), lambda i: (0, 0)),
        pl.BlockSpec((1, H), lambda i: (0, 0)),
        pl.BlockSpec((H, H), lambda i: (0, 0)),
        pl.BlockSpec((1, H), lambda i: (0, 0)),
    ],
    out_specs=pl.BlockSpec((RB, H), lambda i: (i, 0)),
    out_shape=jax.ShapeDtypeStruct((N, H), jnp.float32),
)


def _pool_body(h_ref, b_ref, wl_ref, bl_ref, o_ref, acc_ref):
    i = pl.program_id(0)

    @pl.when(i == 0)
    def _():
        acc_ref[...] = jnp.zeros((G, H), jnp.float32)

    b = b_ref[...].reshape(1, RB)
    gids = lax.broadcasted_iota(jnp.int32, (G, RB), 0)
    mask = (gids == b).astype(jnp.float32)
    acc_ref[...] += jnp.dot(mask, h_ref[...],
                            preferred_element_type=jnp.float32)

    @pl.when(i == NRB - 1)
    def _():
        logits = jnp.dot(acc_ref[...], wl_ref[...],
                         preferred_element_type=jnp.float32) + bl_ref[...]
        m = jnp.max(logits, axis=1, keepdims=True)
        lse = jnp.log(jnp.sum(jnp.exp(logits - m), axis=1, keepdims=True)) + m
        o_ref[...] = logits - lse


_pool_call = pl.pallas_call(
    _pool_body,
    grid=(NRB,),
    in_specs=[
        pl.BlockSpec((RB, H), lambda i: (i, 0)),
        pl.BlockSpec((1, 1, RB), lambda i: (i, 0, 0)),
        pl.BlockSpec((H, OUT), lambda i: (0, 0)),
        pl.BlockSpec((1, OUT), lambda i: (0, 0)),
    ],
    out_specs=pl.BlockSpec((G, OUT), lambda i: (0, 0)),
    out_shape=jax.ShapeDtypeStruct((G, OUT), jnp.float32),
    scratch_shapes=[pltpu.VMEM((G, H), jnp.float32)],
)


def kernel(x, edge_index, batch, W1a, b1a, W1b, b1b, W2a, b2a, W2b, b2b,
           W3a, b3a, W3b, b3b, Wl, bl):
    npad = E_PAD - E
    src = jnp.concatenate(
        [edge_index[0], jnp.zeros((npad,), jnp.int32)]).reshape(NW, NCHUNK, K)
    dst = jnp.concatenate(
        [edge_index[1], jnp.full((npad,), DUMMY, jnp.int32)]).reshape(
            NW, NCHUNK, K)
    zero = jnp.zeros((ZK, D), jnp.float32)
    batch3 = batch.reshape(NRB, 1, RB)

    h = x
    for (Wa, ba, Wb, bb) in ((W1a, b1a, W1b, b1b),
                             (W2a, b2a, W2b, b2b),
                             (W3a, b3a, W3b, b3b)):
        agg = _sc_agg(h, src, dst, zero)
        h = _mlp_call(h, agg, Wa, ba.reshape(1, H), Wb, bb.reshape(1, H))
    return _pool_call(h, batch3, Wl, bl.reshape(1, OUT))


# speedup vs baseline: 2.9384x; 2.9384x over previous
import functools

import jax
import jax.numpy as jnp
from jax import lax
from jax.experimental import pallas as pl
from jax.experimental.pallas import tpu as pltpu
from jax.experimental.pallas import tpu_sc as plsc

N = 10000
E = 320000
D = 128
H = 128
OUT = 64
G = 128

NC = 2
NS = 16
NW = NC * NS
K = 128
NCHUNK = 80
EPW = NCHUNK * K
E_PAD = NW * EPW
HCHUNK = NCHUNK // 2
ACC_N = 10240
DUMMY = ACC_N - 1
RPT = ACC_N // NS
ZK = 128
RB = 1000
NRB = N // RB


_mesh = plsc.VectorSubcoreMesh(core_axis_name="c", subcore_axis_name="s")


@functools.partial(
    pl.kernel,
    out_type=jax.ShapeDtypeStruct((NC, N, D), jnp.float32),
    mesh=_mesh,
    scratch_types=[
        pltpu.VMEM((HCHUNK, K), jnp.int32),
        pltpu.VMEM((HCHUNK, K), jnp.int32),
        pltpu.VMEM((K, D), jnp.float32),
        pltpu.VMEM((K, D), jnp.float32),
        pltpu.VMEM_SHARED((ACC_N, D), jnp.float32),
        pltpu.SemaphoreType.DMA,
        pltpu.SemaphoreType.DMA,
    ],
)
def _sc_agg(x_hbm, src_hbm, dst_hbm, zero_hbm, out_hbm,
            src_v, dst_v, rows_v, rows2_v, acc_sh, sem, sem2):
    c = lax.axis_index("c")
    s = lax.axis_index("s")
    wid = c * NS + s

    pltpu.sync_copy(zero_hbm, rows_v)
    for t in range(RPT // ZK):
        pltpu.sync_copy(rows_v, acc_sh.at[pl.ds(s * RPT + t * ZK, ZK)])
    plsc.subcore_barrier()

    rowsA = rows_v
    rowsB = rows2_v

    def chunk2(jj, carry):
        j0 = jj * 2
        j1 = j0 + 1
        pltpu.async_copy(x_hbm.at[src_v.at[j0]], rowsA, sem).wait()
        pltpu.sync_copy(rowsA, acc_sh.at[dst_v.at[j0]], add=True)
        pltpu.async_copy(x_hbm.at[src_v.at[j1]], rowsB, sem2).wait()
        pltpu.sync_copy(rowsB, acc_sh.at[dst_v.at[j1]], add=True)
        return carry

    for half in range(2):
        pltpu.sync_copy(src_hbm.at[wid].at[pl.ds(half * HCHUNK, HCHUNK)],
                        src_v)
        pltpu.sync_copy(dst_hbm.at[wid].at[pl.ds(half * HCHUNK, HCHUNK)],
                        dst_v)
        lax.fori_loop(0, HCHUNK // 2, chunk2, 0)
    plsc.subcore_barrier()

    @pl.when(s < NS - 1)
    def _():
        pltpu.sync_copy(acc_sh.at[pl.ds(s * RPT, RPT)],
                        out_hbm.at[c].at[pl.ds(s * RPT, RPT)])

    @pl.when(s == NS - 1)
    def _():
        tail = N - (NS - 1) * RPT
        pltpu.sync_copy(acc_sh.at[pl.ds((NS - 1) * RPT, tail)],
                        out_hbm.at[c].at[pl.ds((NS - 1) * RPT, tail)])


def _mlp_body(x_ref, agg_ref, wa_ref, ba_ref, wb_ref, bb_ref, o_ref):
    h = x_ref[...] + agg_ref[0] + agg_ref[1]
    h = jnp.maximum(
        jnp.dot(h, wa_ref[...], preferred_element_type=jnp.float32)
        + ba_ref[...], 0.0)
    h = jnp.maximum(
        jnp.dot(h, wb_ref[...], preferred_element_type=jnp.float32)
        + bb_ref[...], 0.0)
    o_ref[...] = h


_mlp_call = pl.pallas_call(
    _mlp_body,
    grid=(NRB,),
    in_specs=[
        pl.BlockSpec((RB, D), lambda i: (i, 0)),
        pl.BlockSpec((NC, RB, D), lambda i: (0, i, 0)),
        pl.BlockSpec((D, H), lambda i: (0, 0)),
        pl.BlockSpec((1, H), lambda i: (0, 0)),
        pl.BlockSpec((H, H), lambda i: (0, 0)),
        pl.BlockSpec((1, H), lambda i: (0, 0)),
    ],
    out_specs=pl.BlockSpec((RB, H), lambda i: (i, 0)),
    out_shape=jax.ShapeDtypeStruct((N, H), jnp.float32),
)


def _pool_body(h_ref, b_ref, wl_ref, bl_ref, o_ref, acc_ref):
    i = pl.program_id(0)

    @pl.when(i == 0)
    def _():
        acc_ref[...] = jnp.zeros((G, H), jnp.float32)

    b = b_ref[...].reshape(1, RB)
    gids = lax.broadcasted_iota(jnp.int32, (G, RB), 0)
    mask = (gids == b).astype(jnp.float32)
    acc_ref[...] += jnp.dot(mask, h_ref[...],
                            preferred_element_type=jnp.float32)

    @pl.when(i == NRB - 1)
    def _():
        logits = jnp.dot(acc_ref[...], wl_ref[...],
                         preferred_element_type=jnp.float32) + bl_ref[...]
        m = jnp.max(logits, axis=1, keepdims=True)
        lse = jnp.log(jnp.sum(jnp.exp(logits - m), axis=1, keepdims=True)) + m
        o_ref[...] = logits - lse


_pool_call = pl.pallas_call(
    _pool_body,
    grid=(NRB,),
    in_specs=[
        pl.BlockSpec((RB, H), lambda i: (i, 0)),
        pl.BlockSpec((1, 1, RB), lambda i: (i, 0, 0)),
        pl.BlockSpec((H, OUT), lambda i: (0, 0)),
        pl.BlockSpec((1, OUT), lambda i: (0, 0)),
    ],
    out_specs=pl.BlockSpec((G, OUT), lambda i: (0, 0)),
    out_shape=jax.ShapeDtypeStruct((G, OUT), jnp.float32),
    scratch_shapes=[pltpu.VMEM((G, H), jnp.float32)],
)


def kernel(x, edge_index, batch, W1a, b1a, W1b, b1b, W2a, b2a, W2b, b2b,
           W3a, b3a, W3b, b3b, Wl, bl):
    npad = E_PAD - E
    pad_i = jnp.arange(npad, dtype=jnp.int32)
    src = jnp.concatenate(
        [edge_index[0], pad_i % N]).reshape(NW, NCHUNK, K)
    dst = jnp.concatenate(
        [edge_index[1], N + pad_i % (ACC_N - N)]).reshape(NW, NCHUNK, K)
    zero = jnp.zeros((ZK, D), jnp.float32)
    batch3 = batch.reshape(NRB, 1, RB)

    h = x
    for (Wa, ba, Wb, bb) in ((W1a, b1a, W1b, b1b),
                             (W2a, b2a, W2b, b2b),
                             (W3a, b3a, W3b, b3b)):
        agg = _sc_agg(h, src, dst, zero)
        h = _mlp_call(h, agg, Wa, ba.reshape(1, H), Wb, bb.reshape(1, H))
    return _pool_call(h, batch3, Wl, bl.reshape(1, OUT))


# speedup vs baseline: 3.3653x; 1.1453x over previous
import functools

import jax
import jax.numpy as jnp
from jax import lax
from jax.experimental import pallas as pl
from jax.experimental.pallas import tpu as pltpu
from jax.experimental.pallas import tpu_sc as plsc

N = 10000
E = 320000
D = 128
H = 128
OUT = 64
G = 128

NC = 2
NS = 16
NW = NC * NS
K = 128
NCHUNK = 80
EPW = NCHUNK * K
E_PAD = NW * EPW
HCHUNK = NCHUNK // 2
ACC_N = 10240
DUMMY = ACC_N - 1
RPT = ACC_N // NS
ZK = 128
RB = 1000
NRB = N // RB


_mesh = plsc.VectorSubcoreMesh(core_axis_name="c", subcore_axis_name="s")


@functools.partial(
    pl.kernel,
    out_type=jax.ShapeDtypeStruct((NC, N, D), jnp.float32),
    mesh=_mesh,
    scratch_types=[
        pltpu.VMEM((HCHUNK, K), jnp.int32),
        pltpu.VMEM((HCHUNK, K), jnp.int32),
        pltpu.VMEM((K, D), jnp.float32),
        pltpu.VMEM((K, D), jnp.float32),
        pltpu.VMEM_SHARED((ACC_N, D), jnp.float32),
        pltpu.SemaphoreType.DMA,
        pltpu.SemaphoreType.DMA,
    ],
)
def _sc_agg(x_hbm, src_hbm, dst_hbm, zero_hbm, out_hbm,
            src_v, dst_v, rows_v, rows2_v, acc_sh, sem, sem2):
    c = lax.axis_index("c")
    s = lax.axis_index("s")
    wid = c * NS + s

    pltpu.sync_copy(zero_hbm, rows_v)
    for t in range(RPT // ZK):
        pltpu.sync_copy(rows_v, acc_sh.at[pl.ds(s * RPT + t * ZK, ZK)])
    plsc.subcore_barrier()

    rowsA = rows_v
    rowsB = rows2_v

    def chunk2(jj, carry):
        j0 = jj * 2
        j1 = j0 + 1
        cpA = pltpu.async_copy(x_hbm.at[src_v.at[j0]], rowsA, sem)
        cpB = pltpu.async_copy(x_hbm.at[src_v.at[j1]], rowsB, sem2)
        cpA.wait()
        pltpu.sync_copy(rowsA, acc_sh.at[dst_v.at[j0]], add=True)
        cpB.wait()
        pltpu.sync_copy(rowsB, acc_sh.at[dst_v.at[j1]], add=True)
        return carry

    for half in range(2):
        pltpu.sync_copy(src_hbm.at[wid].at[pl.ds(half * HCHUNK, HCHUNK)],
                        src_v)
        pltpu.sync_copy(dst_hbm.at[wid].at[pl.ds(half * HCHUNK, HCHUNK)],
                        dst_v)
        lax.fori_loop(0, HCHUNK // 2, chunk2, 0)
    plsc.subcore_barrier()

    @pl.when(s < NS - 1)
    def _():
        pltpu.sync_copy(acc_sh.at[pl.ds(s * RPT, RPT)],
                        out_hbm.at[c].at[pl.ds(s * RPT, RPT)])

    @pl.when(s == NS - 1)
    def _():
        tail = N - (NS - 1) * RPT
        pltpu.sync_copy(acc_sh.at[pl.ds((NS - 1) * RPT, tail)],
                        out_hbm.at[c].at[pl.ds((NS - 1) * RPT, tail)])


def _mlp_body(x_ref, agg_ref, wa_ref, ba_ref, wb_ref, bb_ref, o_ref):
    h = x_ref[...] + agg_ref[0] + agg_ref[1]
    h = jnp.maximum(
        jnp.dot(h, wa_ref[...], preferred_element_type=jnp.float32)
        + ba_ref[...], 0.0)
    h = jnp.maximum(
        jnp.dot(h, wb_ref[...], preferred_element_type=jnp.float32)
        + bb_ref[...], 0.0)
    o_ref[...] = h


_mlp_call = pl.pallas_call(
    _mlp_body,
    grid=(NRB,),
    in_specs=[
        pl.BlockSpec((RB, D), lambda i: (i, 0)),
        pl.BlockSpec((NC, RB, D), lambda i: (0, i, 0)),
        pl.BlockSpec((D, H), lambda i: (0, 0)),
        pl.BlockSpec((1, H), lambda i: (0, 0)),
        pl.BlockSpec((H, H), lambda i: (0, 0)),
        pl.BlockSpec((1, H), lambda i: (0, 0)),
    ],
    out_specs=pl.BlockSpec((RB, H), lambda i: (i, 0)),
    out_shape=jax.ShapeDtypeStruct((N, H), jnp.float32),
)


def _pool_body(h_ref, b_ref, wl_ref, bl_ref, o_ref, acc_ref):
    i = pl.program_id(0)

    @pl.when(i == 0)
    def _():
        acc_ref[...] = jnp.zeros((G, H), jnp.float32)

    b = b_ref[...].reshape(1, RB)
    gids = lax.broadcasted_iota(jnp.int32, (G, RB), 0)
    mask = (gids == b).astype(jnp.float32)
    acc_ref[...] += jnp.dot(mask, h_ref[...],
                            preferred_element_type=jnp.float32)

    @pl.when(i == NRB - 1)
    def _():
        logits = jnp.dot(acc_ref[...], wl_ref[...],
                         preferred_element_type=jnp.float32) + bl_ref[...]
        m = jnp.max(logits, axis=1, keepdims=True)
        lse = jnp.log(jnp.sum(jnp.exp(logits - m), axis=1, keepdims=True)) + m
        o_ref[...] = logits - lse


_pool_call = pl.pallas_call(
    _pool_body,
    grid=(NRB,),
    in_specs=[
        pl.BlockSpec((RB, H), lambda i: (i, 0)),
        pl.BlockSpec((1, 1, RB), lambda i: (i, 0, 0)),
        pl.BlockSpec((H, OUT), lambda i: (0, 0)),
        pl.BlockSpec((1, OUT), lambda i: (0, 0)),
    ],
    out_specs=pl.BlockSpec((G, OUT), lambda i: (0, 0)),
    out_shape=jax.ShapeDtypeStruct((G, OUT), jnp.float32),
    scratch_shapes=[pltpu.VMEM((G, H), jnp.float32)],
)


def kernel(x, edge_index, batch, W1a, b1a, W1b, b1b, W2a, b2a, W2b, b2b,
           W3a, b3a, W3b, b3b, Wl, bl):
    npad = E_PAD - E
    pad_i = jnp.arange(npad, dtype=jnp.int32)
    src = jnp.concatenate(
        [edge_index[0], pad_i % N]).reshape(NW, NCHUNK, K)
    dst = jnp.concatenate(
        [edge_index[1], N + pad_i % (ACC_N - N)]).reshape(NW, NCHUNK, K)
    zero = jnp.zeros((ZK, D), jnp.float32)
    batch3 = batch.reshape(NRB, 1, RB)

    h = x
    for (Wa, ba, Wb, bb) in ((W1a, b1a, W1b, b1b),
                             (W2a, b2a, W2b, b2b),
                             (W3a, b3a, W3b, b3b)):
        agg = _sc_agg(h, src, dst, zero)
        h = _mlp_call(h, agg, Wa, ba.reshape(1, H), Wb, bb.reshape(1, H))
    return _pool_call(h, batch3, Wl, bl.reshape(1, OUT))


# speedup vs baseline: 4.3819x; 1.3021x over previous
import functools

import jax
import jax.numpy as jnp
from jax import lax
from jax.experimental import pallas as pl
from jax.experimental.pallas import tpu as pltpu
from jax.experimental.pallas import tpu_sc as plsc

N = 10000
E = 320000
D = 128
H = 128
OUT = 64
G = 128

NC = 2
NS = 16
NW = NC * NS
K = 128
NCHUNK = 80
EPW = NCHUNK * K
E_PAD = NW * EPW
HCHUNK = NCHUNK // 2
ACC_N = 10240
DUMMY = ACC_N - 1
RPT = ACC_N // NS
ZK = 128
RB = 1000
NRB = N // RB


_mesh = plsc.VectorSubcoreMesh(core_axis_name="c", subcore_axis_name="s")


@functools.partial(
    pl.kernel,
    out_type=jax.ShapeDtypeStruct((NC, N, D), jnp.float32),
    mesh=_mesh,
    scratch_types=[
        pltpu.VMEM((HCHUNK, K), jnp.int32),
        pltpu.VMEM((HCHUNK, K), jnp.int32),
        pltpu.VMEM((K, D), jnp.float32),
        pltpu.VMEM((K, D), jnp.float32),
        pltpu.VMEM_SHARED((ACC_N, D), jnp.float32),
        pltpu.SemaphoreType.DMA,
        pltpu.SemaphoreType.DMA,
    ],
)
def _sc_agg(x_hbm, src_hbm, dst_hbm, zero_hbm, out_hbm,
            src_v, dst_v, rows_v, rows2_v, acc_sh, sem, sem2):
    c = lax.axis_index("c")
    s = lax.axis_index("s")
    wid = c * NS + s

    pltpu.sync_copy(zero_hbm, rows_v)
    for t in range(RPT // ZK):
        pltpu.sync_copy(rows_v, acc_sh.at[pl.ds(s * RPT + t * ZK, ZK)])
    plsc.subcore_barrier()

    rowsA = rows_v
    rowsB = rows2_v

    def start_gather(j, buf, s_):
        pltpu.async_copy(x_hbm.at[src_v.at[j]], buf, s_)

    def wait_gather(buf, s_):
        pltpu.make_async_copy(x_hbm.at[src_v.at[0]], buf, s_).wait()

    def chunk2(jj, carry):
        j0 = jj * 2
        j1 = j0 + 1
        start_gather(j1, rowsB, sem2)
        wait_gather(rowsA, sem)
        pltpu.sync_copy(rowsA, acc_sh.at[dst_v.at[j0]], add=True)

        @pl.when(j0 + 2 < HCHUNK)
        def _():
            start_gather(j0 + 2, rowsA, sem)

        wait_gather(rowsB, sem2)
        pltpu.sync_copy(rowsB, acc_sh.at[dst_v.at[j1]], add=True)
        return carry

    for half in range(2):
        pltpu.sync_copy(src_hbm.at[wid].at[pl.ds(half * HCHUNK, HCHUNK)],
                        src_v)
        pltpu.sync_copy(dst_hbm.at[wid].at[pl.ds(half * HCHUNK, HCHUNK)],
                        dst_v)
        start_gather(0, rowsA, sem)
        lax.fori_loop(0, HCHUNK // 2, chunk2, 0)
    plsc.subcore_barrier()

    @pl.when(s < NS - 1)
    def _():
        pltpu.sync_copy(acc_sh.at[pl.ds(s * RPT, RPT)],
                        out_hbm.at[c].at[pl.ds(s * RPT, RPT)])

    @pl.when(s == NS - 1)
    def _():
        tail = N - (NS - 1) * RPT
        pltpu.sync_copy(acc_sh.at[pl.ds((NS - 1) * RPT, tail)],
                        out_hbm.at[c].at[pl.ds((NS - 1) * RPT, tail)])


def _mlp_body(x_ref, agg_ref, wa_ref, ba_ref, wb_ref, bb_ref, o_ref):
    h = x_ref[...] + agg_ref[0] + agg_ref[1]
    h = jnp.maximum(
        jnp.dot(h, wa_ref[...], preferred_element_type=jnp.float32)
        + ba_ref[...], 0.0)
    h = jnp.maximum(
        jnp.dot(h, wb_ref[...], preferred_element_type=jnp.float32)
        + bb_ref[...], 0.0)
    o_ref[...] = h


_mlp_call = pl.pallas_call(
    _mlp_body,
    grid=(NRB,),
    in_specs=[
        pl.BlockSpec((RB, D), lambda i: (i, 0)),
        pl.BlockSpec((NC, RB, D), lambda i: (0, i, 0)),
        pl.BlockSpec((D, H), lambda i: (0, 0)),
        pl.BlockSpec((1, H), lambda i: (0, 0)),
        pl.BlockSpec((H, H), lambda i: (0, 0)),
        pl.BlockSpec((1, H), lambda i: (0, 0)),
    ],
    out_specs=pl.BlockSpec((RB, H), lambda i: (i, 0)),
    out_shape=jax.ShapeDtypeStruct((N, H), jnp.float32),
)


def _pool_body(h_ref, b_ref, wl_ref, bl_ref, o_ref, acc_ref):
    i = pl.program_id(0)

    @pl.when(i == 0)
    def _():
        acc_ref[...] = jnp.zeros((G, H), jnp.float32)

    b = b_ref[...].reshape(1, RB)
    gids = lax.broadcasted_iota(jnp.int32, (G, RB), 0)
    mask = (gids == b).astype(jnp.float32)
    acc_ref[...] += jnp.dot(mask, h_ref[...],
                            preferred_element_type=jnp.float32)

    @pl.when(i == NRB - 1)
    def _():
        logits = jnp.dot(acc_ref[...], wl_ref[...],
                         preferred_element_type=jnp.float32) + bl_ref[...]
        m = jnp.max(logits, axis=1, keepdims=True)
        lse = jnp.log(jnp.sum(jnp.exp(logits - m), axis=1, keepdims=True)) + m
        o_ref[...] = logits - lse


_pool_call = pl.pallas_call(
    _pool_body,
    grid=(NRB,),
    in_specs=[
        pl.BlockSpec((RB, H), lambda i: (i, 0)),
        pl.BlockSpec((1, 1, RB), lambda i: (i, 0, 0)),
        pl.BlockSpec((H, OUT), lambda i: (0, 0)),
        pl.BlockSpec((1, OUT), lambda i: (0, 0)),
    ],
    out_specs=pl.BlockSpec((G, OUT), lambda i: (0, 0)),
    out_shape=jax.ShapeDtypeStruct((G, OUT), jnp.float32),
    scratch_shapes=[pltpu.VMEM((G, H), jnp.float32)],
)


def kernel(x, edge_index, batch, W1a, b1a, W1b, b1b, W2a, b2a, W2b, b2b,
           W3a, b3a, W3b, b3b, Wl, bl):
    npad = E_PAD - E
    pad_i = jnp.arange(npad, dtype=jnp.int32)
    src = jnp.concatenate(
        [edge_index[0], pad_i % N]).reshape(NW, NCHUNK, K)
    dst = jnp.concatenate(
        [edge_index[1], N + pad_i % (ACC_N - N)]).reshape(NW, NCHUNK, K)
    zero = jnp.zeros((ZK, D), jnp.float32)
    batch3 = batch.reshape(NRB, 1, RB)

    h = x
    for (Wa, ba, Wb, bb) in ((W1a, b1a, W1b, b1b),
                             (W2a, b2a, W2b, b2b),
                             (W3a, b3a, W3b, b3b)):
        agg = _sc_agg(h, src, dst, zero)
        h = _mlp_call(h, agg, Wa, ba.reshape(1, H), Wb, bb.reshape(1, H))
    return _pool_call(h, batch3, Wl, bl.reshape(1, OUT))


# speedup vs baseline: 4.5199x; 1.0315x over previous
import functools

import jax
import jax.numpy as jnp
from jax import lax
from jax.experimental import pallas as pl
from jax.experimental.pallas import tpu as pltpu
from jax.experimental.pallas import tpu_sc as plsc

N = 10000
E = 320000
D = 128
H = 128
OUT = 64
G = 128

NC = 2
NS = 16
NW = NC * NS
K = 128
NCHUNK = 80
EPW = NCHUNK * K
E_PAD = NW * EPW
HCHUNK = NCHUNK // 2
ACC_N = 10240
DUMMY = ACC_N - 1
RPT = ACC_N // NS
ZK = 128
RB = 1000
NRB = N // RB


_mesh = plsc.VectorSubcoreMesh(core_axis_name="c", subcore_axis_name="s")


@functools.partial(
    pl.kernel,
    out_type=jax.ShapeDtypeStruct((NC, N, D), jnp.float32),
    mesh=_mesh,
    scratch_types=[
        pltpu.VMEM((HCHUNK, K), jnp.int32),
        pltpu.VMEM((HCHUNK, K), jnp.int32),
        pltpu.VMEM((K, D), jnp.float32),
        pltpu.VMEM((K, D), jnp.float32),
        pltpu.VMEM_SHARED((ACC_N, D), jnp.float32),
        pltpu.SemaphoreType.DMA,
        pltpu.SemaphoreType.DMA,
        pltpu.SemaphoreType.DMA,
    ],
)
def _sc_agg(x_hbm, src_hbm, dst_hbm, zero_hbm, out_hbm,
            src_v, dst_v, rows_v, rows2_v, acc_sh, sem, sem2, sem3):
    c = lax.axis_index("c")
    s = lax.axis_index("s")
    wid = c * NS + s

    cp_si = pltpu.async_copy(src_hbm.at[wid].at[pl.ds(0, HCHUNK)], src_v,
                             sem)
    cp_di = pltpu.async_copy(dst_hbm.at[wid].at[pl.ds(0, HCHUNK)], dst_v,
                             sem2)
    pltpu.sync_copy(zero_hbm, rows2_v)
    zcps = [pltpu.async_copy(rows2_v,
                             acc_sh.at[pl.ds(s * RPT + t * ZK, ZK)], sem3)
            for t in range(RPT // ZK)]
    for cp in zcps:
        cp.wait()
    cp_si.wait()
    cp_di.wait()
    plsc.subcore_barrier()

    rowsA = rows_v
    rowsB = rows2_v

    def start_gather(j, buf, s_):
        pltpu.async_copy(x_hbm.at[src_v.at[j]], buf, s_)

    def wait_gather(buf, s_):
        pltpu.make_async_copy(x_hbm.at[src_v.at[0]], buf, s_).wait()

    def chunk2(jj, carry):
        j0 = jj * 2
        j1 = j0 + 1
        start_gather(j1, rowsB, sem2)
        wait_gather(rowsA, sem)
        pltpu.sync_copy(rowsA, acc_sh.at[dst_v.at[j0]], add=True)

        @pl.when(j0 + 2 < HCHUNK)
        def _():
            start_gather(j0 + 2, rowsA, sem)

        wait_gather(rowsB, sem2)
        pltpu.sync_copy(rowsB, acc_sh.at[dst_v.at[j1]], add=True)
        return carry

    for half in range(2):
        if half:
            pltpu.sync_copy(
                src_hbm.at[wid].at[pl.ds(half * HCHUNK, HCHUNK)], src_v)
            pltpu.sync_copy(
                dst_hbm.at[wid].at[pl.ds(half * HCHUNK, HCHUNK)], dst_v)
        start_gather(0, rowsA, sem)
        lax.fori_loop(0, HCHUNK // 2, chunk2, 0)
    plsc.subcore_barrier()

    @pl.when(s < NS - 1)
    def _():
        pltpu.sync_copy(acc_sh.at[pl.ds(s * RPT, RPT)],
                        out_hbm.at[c].at[pl.ds(s * RPT, RPT)])

    @pl.when(s == NS - 1)
    def _():
        tail = N - (NS - 1) * RPT
        pltpu.sync_copy(acc_sh.at[pl.ds((NS - 1) * RPT, tail)],
                        out_hbm.at[c].at[pl.ds((NS - 1) * RPT, tail)])


def _mlp_body(x_ref, agg_ref, wa_ref, ba_ref, wb_ref, bb_ref, o_ref):
    h = x_ref[...] + agg_ref[0] + agg_ref[1]
    h = jnp.maximum(
        jnp.dot(h, wa_ref[...], preferred_element_type=jnp.float32)
        + ba_ref[...], 0.0)
    h = jnp.maximum(
        jnp.dot(h, wb_ref[...], preferred_element_type=jnp.float32)
        + bb_ref[...], 0.0)
    o_ref[...] = h


_mlp_call = pl.pallas_call(
    _mlp_body,
    grid=(NRB,),
    in_specs=[
        pl.BlockSpec((RB, D), lambda i: (i, 0)),
        pl.BlockSpec((NC, RB, D), lambda i: (0, i, 0)),
        pl.BlockSpec((D, H), lambda i: (0, 0)),
        pl.BlockSpec((1, H), lambda i: (0, 0)),
        pl.BlockSpec((H, H), lambda i: (0, 0)),
        pl.BlockSpec((1, H), lambda i: (0, 0)),
    ],
    out_specs=pl.BlockSpec((RB, H), lambda i: (i, 0)),
    out_shape=jax.ShapeDtypeStruct((N, H), jnp.float32),
)


def _mlp_pool_body(x_ref, agg_ref, wa_ref, ba_ref, wb_ref, bb_ref,
                   b_ref, wl_ref, bl_ref, o_ref, acc_ref):
    i = pl.program_id(0)

    @pl.when(i == 0)
    def _():
        acc_ref[...] = jnp.zeros((G, H), jnp.float32)

    h = x_ref[...] + agg_ref[0] + agg_ref[1]
    h = jnp.maximum(
        jnp.dot(h, wa_ref[...], preferred_element_type=jnp.float32)
        + ba_ref[...], 0.0)
    h = jnp.maximum(
        jnp.dot(h, wb_ref[...], preferred_element_type=jnp.float32)
        + bb_ref[...], 0.0)

    b = b_ref[...].reshape(1, RB)
    gids = lax.broadcasted_iota(jnp.int32, (G, RB), 0)
    mask = (gids == b).astype(jnp.float32)
    acc_ref[...] += jnp.dot(mask, h, preferred_element_type=jnp.float32)

    @pl.when(i == NRB - 1)
    def _():
        logits = jnp.dot(acc_ref[...], wl_ref[...],
                         preferred_element_type=jnp.float32) + bl_ref[...]
        m = jnp.max(logits, axis=1, keepdims=True)
        lse = jnp.log(jnp.sum(jnp.exp(logits - m), axis=1, keepdims=True)) + m
        o_ref[...] = logits - lse


_mlp_pool_call = pl.pallas_call(
    _mlp_pool_body,
    grid=(NRB,),
    in_specs=[
        pl.BlockSpec((RB, D), lambda i: (i, 0)),
        pl.BlockSpec((NC, RB, D), lambda i: (0, i, 0)),
        pl.BlockSpec((D, H), lambda i: (0, 0)),
        pl.BlockSpec((1, H), lambda i: (0, 0)),
        pl.BlockSpec((H, H), lambda i: (0, 0)),
        pl.BlockSpec((1, H), lambda i: (0, 0)),
        pl.BlockSpec((1, 1, RB), lambda i: (i, 0, 0)),
        pl.BlockSpec((H, OUT), lambda i: (0, 0)),
        pl.BlockSpec((1, OUT), lambda i: (0, 0)),
    ],
    out_specs=pl.BlockSpec((G, OUT), lambda i: (0, 0)),
    out_shape=jax.ShapeDtypeStruct((G, OUT), jnp.float32),
    scratch_shapes=[pltpu.VMEM((G, H), jnp.float32)],
)


def kernel(x, edge_index, batch, W1a, b1a, W1b, b1b, W2a, b2a, W2b, b2b,
           W3a, b3a, W3b, b3b, Wl, bl):
    npad = E_PAD - E
    pad_i = jnp.arange(npad, dtype=jnp.int32)
    src = jnp.concatenate(
        [edge_index[0], pad_i % N]).reshape(NW, NCHUNK, K)
    dst = jnp.concatenate(
        [edge_index[1], N + pad_i % (ACC_N - N)]).reshape(NW, NCHUNK, K)
    zero = jnp.zeros((ZK, D), jnp.float32)
    batch3 = batch.reshape(NRB, 1, RB)

    h = x
    for (Wa, ba, Wb, bb) in ((W1a, b1a, W1b, b1b),
                             (W2a, b2a, W2b, b2b)):
        agg = _sc_agg(h, src, dst, zero)
        h = _mlp_call(h, agg, Wa, ba.reshape(1, H), Wb, bb.reshape(1, H))
    agg = _sc_agg(h, src, dst, zero)
    return _mlp_pool_call(h, agg, W3a, b3a.reshape(1, H), W3b,
                          b3b.reshape(1, H), batch3, Wl, bl.reshape(1, OUT))


# speedup vs baseline: 4.5298x; 1.0022x over previous
import functools

import jax
import jax.numpy as jnp
from jax import lax
from jax.experimental import pallas as pl
from jax.experimental.pallas import tpu as pltpu
from jax.experimental.pallas import tpu_sc as plsc

N = 10000
E = 320000
D = 128
H = 128
OUT = 64
G = 128

NC = 2
NS = 16
NW = NC * NS
K = 128
NCHUNK = 80
EPW = NCHUNK * K
E_PAD = NW * EPW
HCHUNK = NCHUNK // 2
ACC_N = 10240
DUMMY = ACC_N - 1
RPT = ACC_N // NS
ZK = 128
RB = 1000
NRB = N // RB


_mesh = plsc.VectorSubcoreMesh(core_axis_name="c", subcore_axis_name="s")


@functools.partial(
    pl.kernel,
    out_type=jax.ShapeDtypeStruct((NC, N, D), jnp.float32),
    mesh=_mesh,
    scratch_types=[
        pltpu.VMEM((HCHUNK, K), jnp.int32),
        pltpu.VMEM((HCHUNK, K), jnp.int32),
        pltpu.VMEM((K, D), jnp.float32),
        pltpu.VMEM((K, D), jnp.float32),
        pltpu.VMEM_SHARED((ACC_N, D), jnp.float32),
        pltpu.SemaphoreType.DMA,
        pltpu.SemaphoreType.DMA,
        pltpu.SemaphoreType.DMA,
    ],
)
def _sc_agg(x_hbm, src_hbm, dst_hbm, zero_hbm, out_hbm,
            src_v, dst_v, rows_v, rows2_v, acc_sh, sem, sem2, sem3):
    c = lax.axis_index("c")
    s = lax.axis_index("s")
    wid = c * NS + s

    cp_si = pltpu.async_copy(src_hbm.at[wid].at[pl.ds(0, HCHUNK)], src_v,
                             sem)
    cp_di = pltpu.async_copy(dst_hbm.at[wid].at[pl.ds(0, HCHUNK)], dst_v,
                             sem2)
    pltpu.sync_copy(zero_hbm, rows2_v)
    zcps = [pltpu.async_copy(rows2_v,
                             acc_sh.at[pl.ds(s * RPT + t * ZK, ZK)], sem3)
            for t in range(RPT // ZK)]
    for cp in zcps:
        cp.wait()
    cp_si.wait()
    cp_di.wait()
    plsc.subcore_barrier()

    rowsA = rows_v
    rowsB = rows2_v

    HK = K // 2

    def start_gather(j, buf, s_):
        pltpu.async_copy(x_hbm.at[src_v.at[j, pl.ds(0, HK)]],
                         buf.at[pl.ds(0, HK)], s_)
        pltpu.async_copy(x_hbm.at[src_v.at[j, pl.ds(HK, HK)]],
                         buf.at[pl.ds(HK, HK)], s_)

    def wait_gather(buf, s_):
        pltpu.make_async_copy(x_hbm.at[src_v.at[0, pl.ds(0, HK)]],
                              buf.at[pl.ds(0, HK)], s_).wait()
        pltpu.make_async_copy(x_hbm.at[src_v.at[0, pl.ds(0, HK)]],
                              buf.at[pl.ds(HK, HK)], s_).wait()

    def chunk2(jj, carry):
        j0 = jj * 2
        j1 = j0 + 1
        start_gather(j1, rowsB, sem2)
        wait_gather(rowsA, sem)
        pltpu.sync_copy(rowsA, acc_sh.at[dst_v.at[j0]], add=True)

        @pl.when(j0 + 2 < HCHUNK)
        def _():
            start_gather(j0 + 2, rowsA, sem)

        wait_gather(rowsB, sem2)
        pltpu.sync_copy(rowsB, acc_sh.at[dst_v.at[j1]], add=True)
        return carry

    for half in range(2):
        if half:
            pltpu.sync_copy(
                src_hbm.at[wid].at[pl.ds(half * HCHUNK, HCHUNK)], src_v)
            pltpu.sync_copy(
                dst_hbm.at[wid].at[pl.ds(half * HCHUNK, HCHUNK)], dst_v)
        start_gather(0, rowsA, sem)
        lax.fori_loop(0, HCHUNK // 2, chunk2, 0)
    plsc.subcore_barrier()

    @pl.when(s < NS - 1)
    def _():
        pltpu.sync_copy(acc_sh.at[pl.ds(s * RPT, RPT)],
                        out_hbm.at[c].at[pl.ds(s * RPT, RPT)])

    @pl.when(s == NS - 1)
    def _():
        tail = N - (NS - 1) * RPT
        pltpu.sync_copy(acc_sh.at[pl.ds((NS - 1) * RPT, tail)],
                        out_hbm.at[c].at[pl.ds((NS - 1) * RPT, tail)])


def _mlp_body(x_ref, agg_ref, wa_ref, ba_ref, wb_ref, bb_ref, o_ref):
    h = x_ref[...] + agg_ref[0] + agg_ref[1]
    h = jnp.maximum(
        jnp.dot(h, wa_ref[...], preferred_element_type=jnp.float32)
        + ba_ref[...], 0.0)
    h = jnp.maximum(
        jnp.dot(h, wb_ref[...], preferred_element_type=jnp.float32)
        + bb_ref[...], 0.0)
    o_ref[...] = h


_mlp_call = pl.pallas_call(
    _mlp_body,
    grid=(NRB,),
    in_specs=[
        pl.BlockSpec((RB, D), lambda i: (i, 0)),
        pl.BlockSpec((NC, RB, D), lambda i: (0, i, 0)),
        pl.BlockSpec((D, H), lambda i: (0, 0)),
        pl.BlockSpec((1, H), lambda i: (0, 0)),
        pl.BlockSpec((H, H), lambda i: (0, 0)),
        pl.BlockSpec((1, H), lambda i: (0, 0)),
    ],
    out_specs=pl.BlockSpec((RB, H), lambda i: (i, 0)),
    out_shape=jax.ShapeDtypeStruct((N, H), jnp.float32),
)


def _mlp_pool_body(x_ref, agg_ref, wa_ref, ba_ref, wb_ref, bb_ref,
                   b_ref, wl_ref, bl_ref, o_ref, acc_ref):
    i = pl.program_id(0)

    @pl.when(i == 0)
    def _():
        acc_ref[...] = jnp.zeros((G, H), jnp.float32)

    h = x_ref[...] + agg_ref[0] + agg_ref[1]
    h = jnp.maximum(
        jnp.dot(h, wa_ref[...], preferred_element_type=jnp.float32)
        + ba_ref[...], 0.0)
    h = jnp.maximum(
        jnp.dot(h, wb_ref[...], preferred_element_type=jnp.float32)
        + bb_ref[...], 0.0)

    b = b_ref[...].reshape(1, RB)
    gids = lax.broadcasted_iota(jnp.int32, (G, RB), 0)
    mask = (gids == b).astype(jnp.float32)
    acc_ref[...] += jnp.dot(mask, h, preferred_element_type=jnp.float32)

    @pl.when(i == NRB - 1)
    def _():
        logits = jnp.dot(acc_ref[...], wl_ref[...],
                         preferred_element_type=jnp.float32) + bl_ref[...]
        m = jnp.max(logits, axis=1, keepdims=True)
        lse = jnp.log(jnp.sum(jnp.exp(logits - m), axis=1, keepdims=True)) + m
        o_ref[...] = logits - lse


_mlp_pool_call = pl.pallas_call(
    _mlp_pool_body,
    grid=(NRB,),
    in_specs=[
        pl.BlockSpec((RB, D), lambda i: (i, 0)),
        pl.BlockSpec((NC, RB, D), lambda i: (0, i, 0)),
        pl.BlockSpec((D, H), lambda i: (0, 0)),
        pl.BlockSpec((1, H), lambda i: (0, 0)),
        pl.BlockSpec((H, H), lambda i: (0, 0)),
        pl.BlockSpec((1, H), lambda i: (0, 0)),
        pl.BlockSpec((1, 1, RB), lambda i: (i, 0, 0)),
        pl.BlockSpec((H, OUT), lambda i: (0, 0)),
        pl.BlockSpec((1, OUT), lambda i: (0, 0)),
    ],
    out_specs=pl.BlockSpec((G, OUT), lambda i: (0, 0)),
    out_shape=jax.ShapeDtypeStruct((G, OUT), jnp.float32),
    scratch_shapes=[pltpu.VMEM((G, H), jnp.float32)],
)


def kernel(x, edge_index, batch, W1a, b1a, W1b, b1b, W2a, b2a, W2b, b2b,
           W3a, b3a, W3b, b3b, Wl, bl):
    npad = E_PAD - E
    pad_i = jnp.arange(npad, dtype=jnp.int32)
    src = jnp.concatenate(
        [edge_index[0], pad_i % N]).reshape(NW, NCHUNK, K)
    dst = jnp.concatenate(
        [edge_index[1], N + pad_i % (ACC_N - N)]).reshape(NW, NCHUNK, K)
    zero = jnp.zeros((ZK, D), jnp.float32)
    batch3 = batch.reshape(NRB, 1, RB)

    h = x
    for (Wa, ba, Wb, bb) in ((W1a, b1a, W1b, b1b),
                             (W2a, b2a, W2b, b2b)):
        agg = _sc_agg(h, src, dst, zero)
        h = _mlp_call(h, agg, Wa, ba.reshape(1, H), Wb, bb.reshape(1, H))
    agg = _sc_agg(h, src, dst, zero)
    return _mlp_pool_call(h, agg, W3a, b3a.reshape(1, H), W3b,
                          b3b.reshape(1, H), batch3, Wl, bl.reshape(1, OUT))
